# initial kernel scaffold (unmeasured)
import jax
import jax.numpy as jnp
from jax import lax
from jax.experimental import pallas as pl
from jax.experimental.pallas import tpu as pltpu

N_DEV = 8


def kernel(x, w_mat):
    m, _ = x.shape
    _, n = w_mat.shape
    mc = m // N_DEV
    nh = n // 2

    xb = x.astype(jnp.bfloat16)
    wb = w_mat.astype(jnp.bfloat16)

    def body(x_ref, w_ref, out_ref, comm_ref, send_sems, recv_sems, credit_sems):
        my = lax.axis_index("i")
        right = lax.rem(my + 1, N_DEV)
        left = lax.rem(my + N_DEV - 1, N_DEV)

        barrier = pltpu.get_barrier_semaphore()
        for nbr in (left, right):
            pl.semaphore_signal(
                barrier, inc=1, device_id=(nbr,),
                device_id_type=pl.DeviceIdType.MESH,
            )
        pl.semaphore_wait(barrier, 2)

        def partial(c, d):
            xc = x_ref[pl.ds(c * mc, mc), :]
            wc = w_ref[:, pl.ds(d * nh, nh)]
            return jnp.dot(xc, wc, preferred_element_type=jnp.float32)

        comm_ref[0, 0] = partial(left, 0).astype(jnp.bfloat16)
        comm_ref[1, 0] = partial(right, 1).astype(jnp.bfloat16)

        for s in range(N_DEV - 1):
            slot_s = s % 2
            slot_r = (s + 1) % 2
            rdmas = []
            for d, dst in ((0, right), (1, left)):
                if s >= 1:
                    pl.semaphore_wait(credit_sems.at[d], 1)
                rdma = pltpu.make_async_remote_copy(
                    src_ref=comm_ref.at[d, slot_s],
                    dst_ref=comm_ref.at[d, slot_r],
                    send_sem=send_sems.at[d, slot_s],
                    recv_sem=recv_sems.at[d, slot_r],
                    device_id=(dst,),
                    device_id_type=pl.DeviceIdType.MESH,
                )
                rdma.start()
                rdmas.append(rdma)

            c0 = lax.rem(my + 2 * N_DEV - 2 - s, N_DEV)
            c1 = lax.rem(my + 2 + s, N_DEV)
            last = s == N_DEV - 2

            p0 = partial(c0, 0)
            rdmas[0].wait_recv()
            acc0 = comm_ref[0, slot_r].astype(jnp.float32) + p0
            if last:
                out_ref[:, pl.ds(0, nh)] = jnp.maximum(acc0, 0.0)
            else:
                comm_ref[0, slot_r] = acc0.astype(jnp.bfloat16)

            p1 = partial(c1, 1)
            rdmas[1].wait_recv()
            acc1 = comm_ref[1, slot_r].astype(jnp.float32) + p1
            if last:
                out_ref[:, pl.ds(nh, nh)] = jnp.maximum(acc1, 0.0)
            else:
                comm_ref[1, slot_r] = acc1.astype(jnp.bfloat16)

            for d, upstream in ((0, left), (1, right)):
                rdmas[d].wait_send()
                if not last:
                    pl.semaphore_signal(
                        credit_sems.at[d], inc=1, device_id=(upstream,),
                        device_id_type=pl.DeviceIdType.MESH,
                    )

    return pl.pallas_call(
        body,
        out_shape=jax.ShapeDtypeStruct((mc, n), jnp.float32),
        in_specs=[
            pl.BlockSpec(memory_space=pltpu.VMEM),
            pl.BlockSpec(memory_space=pltpu.VMEM),
        ],
        out_specs=pl.BlockSpec(memory_space=pltpu.VMEM),
        scratch_shapes=[
            pltpu.VMEM((2, 2, mc, nh), jnp.bfloat16),
            pltpu.SemaphoreType.DMA((2, 2)),
            pltpu.SemaphoreType.DMA((2, 2)),
            pltpu.SemaphoreType.REGULAR((2,)),
        ],
        compiler_params=pltpu.CompilerParams(collective_id=0),
    )(xb, wb)


# baseline (device time: 410923 ns/iter reference)
import jax
import jax.numpy as jnp
from jax import lax
from jax.experimental import pallas as pl
from jax.experimental.pallas import tpu as pltpu

N_DEV = 8
NT = 2048


def kernel(x, w_mat):
    m, _ = x.shape
    _, n = w_mat.shape
    mc = m // N_DEV
    nh = n // 2
    nt = min(NT, nh)

    xb = x.astype(jnp.bfloat16)
    wb = w_mat.astype(jnp.bfloat16)

    def body(x_ref, w_ref, out_ref, comm_ref, p_ref, send_sems, recv_sems,
             credit_sems):
        my = lax.axis_index("i")
        right = lax.rem(my + 1, N_DEV)
        left = lax.rem(my + N_DEV - 1, N_DEV)

        barrier = pltpu.get_barrier_semaphore()
        for nbr in (left, right):
            pl.semaphore_signal(
                barrier, inc=1, device_id=(nbr,),
                device_id_type=pl.DeviceIdType.MESH,
            )
        pl.semaphore_wait(barrier, 2)

        def strip_dot(c, d, t):
            xc = x_ref[pl.ds(c * mc, mc), :]
            wc = w_ref[:, pl.ds(d * nh + t * nt, nt)]
            return jnp.dot(xc, wc, preferred_element_type=jnp.float32)

        def compute_p(c, d):
            for t in range(nh // nt):
                p_ref[:, pl.ds(t * nt, nt)] = strip_dot(c, d, t)

        def accum(d, slot, last):
            for t in range(nh // nt):
                seg = (comm_ref[d, slot, :, pl.ds(t * nt, nt)].astype(jnp.float32)
                       + p_ref[:, pl.ds(t * nt, nt)])
                if last:
                    out_ref[:, pl.ds(d * nh + t * nt, nt)] = jnp.maximum(seg, 0.0).astype(jnp.bfloat16)
                else:
                    comm_ref[d, slot, :, pl.ds(t * nt, nt)] = seg.astype(
                        jnp.bfloat16)

        for d, c in ((0, left), (1, right)):
            for t in range(nh // nt):
                comm_ref[d, 0, :, pl.ds(t * nt, nt)] = strip_dot(c, d, t).astype(
                    jnp.bfloat16)

        for s in range(N_DEV - 1):
            slot_s = s % 2
            slot_r = (s + 1) % 2
            rdmas = []
            for d, dst in ((0, right), (1, left)):
                if s >= 1:
                    pl.semaphore_wait(credit_sems.at[d], 1)
                rdma = pltpu.make_async_remote_copy(
                    src_ref=comm_ref.at[d, slot_s],
                    dst_ref=comm_ref.at[d, slot_r],
                    send_sem=send_sems.at[d, slot_s],
                    recv_sem=recv_sems.at[d, slot_r],
                    device_id=(dst,),
                    device_id_type=pl.DeviceIdType.MESH,
                )
                rdma.start()
                rdmas.append(rdma)

            c0 = lax.rem(my + 2 * N_DEV - 2 - s, N_DEV)
            c1 = lax.rem(my + 2 + s, N_DEV)
            last = s == N_DEV - 2

            compute_p(c0, 0)
            rdmas[0].wait_recv()
            accum(0, slot_r, last)

            compute_p(c1, 1)
            rdmas[1].wait_recv()
            accum(1, slot_r, last)

            for d, upstream in ((0, left), (1, right)):
                rdmas[d].wait_send()
                if not last:
                    pl.semaphore_signal(
                        credit_sems.at[d], inc=1, device_id=(upstream,),
                        device_id_type=pl.DeviceIdType.MESH,
                    )

    return pl.pallas_call(
        body,
        out_shape=jax.ShapeDtypeStruct((mc, n), jnp.bfloat16),
        in_specs=[
            pl.BlockSpec(memory_space=pltpu.VMEM),
            pl.BlockSpec(memory_space=pltpu.VMEM),
        ],
        out_specs=pl.BlockSpec(memory_space=pltpu.VMEM),
        scratch_shapes=[
            pltpu.VMEM((2, 2, mc, nh), jnp.bfloat16),
            pltpu.VMEM((mc, nh), jnp.float32),
            pltpu.SemaphoreType.DMA((2, 2)),
            pltpu.SemaphoreType.DMA((2, 2)),
            pltpu.SemaphoreType.REGULAR((2,)),
        ],
        compiler_params=pltpu.CompilerParams(
            collective_id=0,
            vmem_limit_bytes=58 * 1024 * 1024,
        ),
    )(xb, wb)


# device time: 358117 ns/iter; 1.1475x vs baseline; 1.1475x over previous
import jax
import jax.numpy as jnp
from jax import lax
from jax.experimental import pallas as pl
from jax.experimental.pallas import tpu as pltpu

N_DEV = 8
NS = 2


def kernel(x, w_mat):
    m, _ = x.shape
    _, n = w_mat.shape
    mc = m // N_DEV
    nh = n // 2
    ns = nh // NS

    xb = x.astype(jnp.bfloat16)
    wb = w_mat.astype(jnp.bfloat16)

    def body(x_ref, w_ref, out_ref, comm_ref, send_sems, recv_sems,
             credit_sems):
        my = lax.axis_index("i")
        right = lax.rem(my + 1, N_DEV)
        left = lax.rem(my + N_DEV - 1, N_DEV)

        barrier = pltpu.get_barrier_semaphore()
        for nbr in (left, right):
            pl.semaphore_signal(
                barrier, inc=1, device_id=(nbr,),
                device_id_type=pl.DeviceIdType.MESH,
            )
        pl.semaphore_wait(barrier, 2)

        rings = [(d, k) for k in range(NS) for d in (0, 1)]

        def dst_of(d):
            return right if d == 0 else left

        def src_of(d):
            return left if d == 0 else right

        def strip_dot(c, d, k):
            xc = x_ref[pl.ds(c * mc, mc), :]
            wc = w_ref[:, pl.ds(d * nh + k * ns, ns)]
            return jnp.dot(xc, wc, preferred_element_type=jnp.float32)

        def rdma(d, k, h):
            return pltpu.make_async_remote_copy(
                src_ref=comm_ref.at[d, k, h % 2],
                dst_ref=comm_ref.at[d, k, (h + 1) % 2],
                send_sem=send_sems.at[d, k, h % 2],
                recv_sem=recv_sems.at[d, k, (h + 1) % 2],
                device_id=(dst_of(d),),
                device_id_type=pl.DeviceIdType.MESH,
            )

        for d, k in rings:
            c = left if d == 0 else right
            comm_ref[d, k, 0] = strip_dot(c, d, k).astype(jnp.bfloat16)
        for d, k in rings:
            rdma(d, k, 0).start()

        for h in range(N_DEV - 1):
            last = h == N_DEV - 2
            for d, k in rings:
                slot_r = (h + 1) % 2
                desc = rdma(d, k, h)
                desc.wait_recv()
                desc.wait_send()
                if not last:
                    pl.semaphore_signal(
                        credit_sems.at[d, k], inc=1,
                        device_id=(src_of(d),),
                        device_id_type=pl.DeviceIdType.MESH,
                    )
                c = lax.rem(my + 2 * N_DEV - 2 - h, N_DEV) if d == 0 else (
                    lax.rem(my + 2 + h, N_DEV))
                seg = (comm_ref[d, k, slot_r].astype(jnp.float32)
                       + strip_dot(c, d, k))
                if last:
                    out_ref[:, pl.ds(d * nh + k * ns, ns)] = jnp.maximum(
                        seg, 0.0).astype(jnp.bfloat16)
                else:
                    comm_ref[d, k, slot_r] = seg.astype(jnp.bfloat16)
                    pl.semaphore_wait(credit_sems.at[d, k], 1)
                    rdma(d, k, h + 1).start()

    return pl.pallas_call(
        body,
        out_shape=jax.ShapeDtypeStruct((mc, n), jnp.bfloat16),
        in_specs=[
            pl.BlockSpec(memory_space=pltpu.VMEM),
            pl.BlockSpec(memory_space=pltpu.VMEM),
        ],
        out_specs=pl.BlockSpec(memory_space=pltpu.VMEM),
        scratch_shapes=[
            pltpu.VMEM((2, NS, 2, mc, ns), jnp.bfloat16),
            pltpu.SemaphoreType.DMA((2, NS, 2)),
            pltpu.SemaphoreType.DMA((2, NS, 2)),
            pltpu.SemaphoreType.REGULAR((2, NS)),
        ],
        compiler_params=pltpu.CompilerParams(
            collective_id=0,
            vmem_limit_bytes=52 * 1024 * 1024,
        ),
    )(xb, wb)


# device time: 354406 ns/iter; 1.1595x vs baseline; 1.0105x over previous
import jax
import jax.numpy as jnp
from jax import lax
from jax.experimental import pallas as pl
from jax.experimental.pallas import tpu as pltpu

N_DEV = 8
NS = 4


def kernel(x, w_mat):
    m, _ = x.shape
    _, n = w_mat.shape
    mc = m // N_DEV
    nh = n // 2
    ns = nh // NS

    xb = x.astype(jnp.bfloat16)
    wb = w_mat.astype(jnp.bfloat16)

    def body(x_ref, w_ref, out_ref, comm_ref, send_sems, recv_sems,
             credit_sems):
        my = lax.axis_index("i")
        right = lax.rem(my + 1, N_DEV)
        left = lax.rem(my + N_DEV - 1, N_DEV)

        barrier = pltpu.get_barrier_semaphore()
        for nbr in (left, right):
            pl.semaphore_signal(
                barrier, inc=1, device_id=(nbr,),
                device_id_type=pl.DeviceIdType.MESH,
            )
        pl.semaphore_wait(barrier, 2)

        rings = [(d, k) for k in range(NS) for d in (0, 1)]

        def dst_of(d):
            return right if d == 0 else left

        def src_of(d):
            return left if d == 0 else right

        def strip_dot(c, d, k):
            xc = x_ref[pl.ds(c * mc, mc), :]
            wc = w_ref[:, pl.ds(d * nh + k * ns, ns)]
            return jnp.dot(xc, wc, preferred_element_type=jnp.float32)

        def rdma(d, k, h):
            return pltpu.make_async_remote_copy(
                src_ref=comm_ref.at[d, k, h % 2],
                dst_ref=comm_ref.at[d, k, (h + 1) % 2],
                send_sem=send_sems.at[d, k, h % 2],
                recv_sem=recv_sems.at[d, k, (h + 1) % 2],
                device_id=(dst_of(d),),
                device_id_type=pl.DeviceIdType.MESH,
            )

        for d, k in rings:
            c = left if d == 0 else right
            comm_ref[d, k, 0] = strip_dot(c, d, k).astype(jnp.bfloat16)
            rdma(d, k, 0).start()

        for h in range(N_DEV - 1):
            last = h == N_DEV - 2
            for d, k in rings:
                slot_r = (h + 1) % 2
                desc = rdma(d, k, h)
                desc.wait_recv()
                desc.wait_send()
                if not last:
                    pl.semaphore_signal(
                        credit_sems.at[d, k], inc=1,
                        device_id=(src_of(d),),
                        device_id_type=pl.DeviceIdType.MESH,
                    )
                c = lax.rem(my + 2 * N_DEV - 2 - h, N_DEV) if d == 0 else (
                    lax.rem(my + 2 + h, N_DEV))
                seg = (comm_ref[d, k, slot_r].astype(jnp.float32)
                       + strip_dot(c, d, k))
                if last:
                    out_ref[:, pl.ds(d * nh + k * ns, ns)] = jnp.maximum(
                        seg, 0.0).astype(jnp.bfloat16)
                else:
                    comm_ref[d, k, slot_r] = seg.astype(jnp.bfloat16)
                    pl.semaphore_wait(credit_sems.at[d, k], 1)
                    rdma(d, k, h + 1).start()

    return pl.pallas_call(
        body,
        out_shape=jax.ShapeDtypeStruct((mc, n), jnp.bfloat16),
        in_specs=[
            pl.BlockSpec(memory_space=pltpu.VMEM),
            pl.BlockSpec(memory_space=pltpu.VMEM),
        ],
        out_specs=pl.BlockSpec(memory_space=pltpu.VMEM),
        scratch_shapes=[
            pltpu.VMEM((2, NS, 2, mc, ns), jnp.bfloat16),
            pltpu.SemaphoreType.DMA((2, NS, 2)),
            pltpu.SemaphoreType.DMA((2, NS, 2)),
            pltpu.SemaphoreType.REGULAR((2, NS)),
        ],
        compiler_params=pltpu.CompilerParams(
            collective_id=0,
            vmem_limit_bytes=52 * 1024 * 1024,
        ),
    )(xb, wb)


# device time: 342951 ns/iter; 1.1982x vs baseline; 1.0334x over previous
import jax
import jax.numpy as jnp
from jax import lax
from jax.experimental import pallas as pl
from jax.experimental.pallas import tpu as pltpu

N_DEV = 8
NS = 4


def kernel(x, w_mat):
    m, _ = x.shape
    _, n = w_mat.shape
    mc = m // N_DEV
    nh = n // 2
    ns = nh // NS

    def body(x_ref, w_ref, out_ref, comm_ref, send_sems, recv_sems,
             credit_sems):
        my = lax.axis_index("i")
        right = lax.rem(my + 1, N_DEV)
        left = lax.rem(my + N_DEV - 1, N_DEV)

        barrier = pltpu.get_barrier_semaphore()
        for nbr in (left, right):
            pl.semaphore_signal(
                barrier, inc=1, device_id=(nbr,),
                device_id_type=pl.DeviceIdType.MESH,
            )
        pl.semaphore_wait(barrier, 2)

        rings = [(d, k) for k in range(NS) for d in (0, 1)]

        def dst_of(d):
            return right if d == 0 else left

        def src_of(d):
            return left if d == 0 else right

        def strip_dot(c, d, k):
            xc = x_ref[pl.ds(c * mc, mc), :].astype(jnp.bfloat16)
            wc = w_ref[:, pl.ds(d * nh + k * ns, ns)].astype(jnp.bfloat16)
            return jnp.dot(xc, wc, preferred_element_type=jnp.float32)

        def rdma(d, k, h):
            return pltpu.make_async_remote_copy(
                src_ref=comm_ref.at[d, k, h % 2],
                dst_ref=comm_ref.at[d, k, (h + 1) % 2],
                send_sem=send_sems.at[d, k, h % 2],
                recv_sem=recv_sems.at[d, k, (h + 1) % 2],
                device_id=(dst_of(d),),
                device_id_type=pl.DeviceIdType.MESH,
            )

        for d, k in rings:
            c = left if d == 0 else right
            comm_ref[d, k, 0] = strip_dot(c, d, k).astype(jnp.bfloat16)
            rdma(d, k, 0).start()

        for h in range(N_DEV - 1):
            last = h == N_DEV - 2
            for d, k in rings:
                slot_r = (h + 1) % 2
                desc = rdma(d, k, h)
                desc.wait_recv()
                desc.wait_send()
                if not last:
                    pl.semaphore_signal(
                        credit_sems.at[d, k], inc=1,
                        device_id=(src_of(d),),
                        device_id_type=pl.DeviceIdType.MESH,
                    )
                c = lax.rem(my + 2 * N_DEV - 2 - h, N_DEV) if d == 0 else (
                    lax.rem(my + 2 + h, N_DEV))
                seg = (comm_ref[d, k, slot_r].astype(jnp.float32)
                       + strip_dot(c, d, k))
                if last:
                    out_ref[:, pl.ds(d * nh + k * ns, ns)] = jnp.maximum(
                        seg, 0.0).astype(jnp.bfloat16)
                else:
                    comm_ref[d, k, slot_r] = seg.astype(jnp.bfloat16)
                    pl.semaphore_wait(credit_sems.at[d, k], 1)
                    rdma(d, k, h + 1).start()

    return pl.pallas_call(
        body,
        out_shape=jax.ShapeDtypeStruct((mc, n), jnp.bfloat16),
        in_specs=[
            pl.BlockSpec(memory_space=pltpu.VMEM),
            pl.BlockSpec(memory_space=pltpu.VMEM),
        ],
        out_specs=pl.BlockSpec(memory_space=pltpu.VMEM),
        scratch_shapes=[
            pltpu.VMEM((2, NS, 2, mc, ns), jnp.bfloat16),
            pltpu.SemaphoreType.DMA((2, NS, 2)),
            pltpu.SemaphoreType.DMA((2, NS, 2)),
            pltpu.SemaphoreType.REGULAR((2, NS)),
        ],
        compiler_params=pltpu.CompilerParams(
            collective_id=0,
            vmem_limit_bytes=58 * 1024 * 1024,
        ),
    )(x, w_mat)


# device time: 342620 ns/iter; 1.1994x vs baseline; 1.0010x over previous
import jax
import jax.numpy as jnp
from jax import lax
from jax.experimental import pallas as pl
from jax.experimental.pallas import tpu as pltpu

N_DEV = 8
NS = 8


def kernel(x, w_mat):
    m, _ = x.shape
    _, n = w_mat.shape
    mc = m // N_DEV
    nh = n // 2
    ns = nh // NS

    def body(x_ref, w_ref, out_ref, comm_ref, send_sems, recv_sems,
             credit_sems):
        my = lax.axis_index("i")
        right = lax.rem(my + 1, N_DEV)
        left = lax.rem(my + N_DEV - 1, N_DEV)

        barrier = pltpu.get_barrier_semaphore()
        for nbr in (left, right):
            pl.semaphore_signal(
                barrier, inc=1, device_id=(nbr,),
                device_id_type=pl.DeviceIdType.MESH,
            )
        pl.semaphore_wait(barrier, 2)

        rings = [(d, k) for k in range(NS) for d in (0, 1)]

        def dst_of(d):
            return right if d == 0 else left

        def src_of(d):
            return left if d == 0 else right

        def strip_dot(c, d, k):
            xc = x_ref[pl.ds(c * mc, mc), :].astype(jnp.bfloat16)
            wc = w_ref[:, pl.ds(d * nh + k * ns, ns)].astype(jnp.bfloat16)
            return jnp.dot(xc, wc, preferred_element_type=jnp.float32)

        def rdma(d, k, h):
            return pltpu.make_async_remote_copy(
                src_ref=comm_ref.at[d, k, h % 2],
                dst_ref=comm_ref.at[d, k, (h + 1) % 2],
                send_sem=send_sems.at[d, k, h % 2],
                recv_sem=recv_sems.at[d, k, (h + 1) % 2],
                device_id=(dst_of(d),),
                device_id_type=pl.DeviceIdType.MESH,
            )

        for d, k in rings:
            c = left if d == 0 else right
            comm_ref[d, k, 0] = strip_dot(c, d, k).astype(jnp.bfloat16)
            rdma(d, k, 0).start()

        for h in range(N_DEV - 1):
            last = h == N_DEV - 2
            for d, k in rings:
                slot_r = (h + 1) % 2
                desc = rdma(d, k, h)
                desc.wait_recv()
                desc.wait_send()
                if not last:
                    pl.semaphore_signal(
                        credit_sems.at[d, k], inc=1,
                        device_id=(src_of(d),),
                        device_id_type=pl.DeviceIdType.MESH,
                    )
                c = lax.rem(my + 2 * N_DEV - 2 - h, N_DEV) if d == 0 else (
                    lax.rem(my + 2 + h, N_DEV))
                seg = (comm_ref[d, k, slot_r].astype(jnp.float32)
                       + strip_dot(c, d, k))
                if last:
                    out_ref[:, pl.ds(d * nh + k * ns, ns)] = jnp.maximum(
                        seg, 0.0).astype(jnp.bfloat16)
                else:
                    comm_ref[d, k, slot_r] = seg.astype(jnp.bfloat16)
                    pl.semaphore_wait(credit_sems.at[d, k], 1)
                    rdma(d, k, h + 1).start()

    return pl.pallas_call(
        body,
        out_shape=jax.ShapeDtypeStruct((mc, n), jnp.bfloat16),
        in_specs=[
            pl.BlockSpec(memory_space=pltpu.VMEM),
            pl.BlockSpec(memory_space=pltpu.VMEM),
        ],
        out_specs=pl.BlockSpec(memory_space=pltpu.VMEM),
        scratch_shapes=[
            pltpu.VMEM((2, NS, 2, mc, ns), jnp.bfloat16),
            pltpu.SemaphoreType.DMA((2, NS, 2)),
            pltpu.SemaphoreType.DMA((2, NS, 2)),
            pltpu.SemaphoreType.REGULAR((2, NS)),
        ],
        compiler_params=pltpu.CompilerParams(
            collective_id=0,
            vmem_limit_bytes=58 * 1024 * 1024,
        ),
    )(x, w_mat)
